# Initial kernel scaffold; baseline (speedup 1.0000x reference)
import jax
import jax.numpy as jnp
from jax.experimental import pallas as pl


def kernel(idx, connections, e1_degrees, embedding_weight):
    raise NotImplementedError("write your pallas kernel here")


# speedup vs baseline: 14.4522x; 14.4522x over previous
import functools

import jax
import jax.numpy as jnp
from jax import lax
from jax.experimental import pallas as pl
from jax.experimental.pallas import tpu as pltpu
from jax.experimental.pallas import tpu_sc as plsc

NC = 2
NS = 16
NW = NC * NS
L = 16
E = 16
UNROLL = 5


@functools.lru_cache(maxsize=None)
def _build(num_ent, d, k, ntot):
    assert d % L == 0 and ntot % (NW * E) == 0 and k % UNROLL == 0
    kp = (k + L - 1) // L * L
    nd = d // L
    nb_ent = ntot // NW
    nch = nb_ent // E

    mesh = plsc.VectorSubcoreMesh(core_axis_name="c", subcore_axis_name="s")

    @functools.partial(
        pl.kernel,
        out_type=jax.ShapeDtypeStruct((ntot, d), jnp.float32),
        mesh=mesh,
        compiler_params=pltpu.CompilerParams(
            use_tc_tiling_on_sc=False, needs_layout_passes=False),
        scratch_types=[
            pltpu.VMEM((nb_ent,), jnp.int32),
            pltpu.VMEM((E, kp), jnp.int32),
            pltpu.VMEM((E, L), jnp.int32),
            pltpu.VMEM((E, d), jnp.float32),
            pltpu.VMEM((k, E, d), jnp.float32),
            pltpu.VMEM((nb_ent, d), jnp.float32),
            pltpu.SemaphoreType.DMA,
            pltpu.SemaphoreType.DMA,
        ],
    )
    def onehop(idx_hbm, conn_hbm, deg_hbm, emb_hbm, out_hbm,
               ent_v, conn_v, deg_v, self_v, nb_v, out_all, sem_a, sem_n):
        c = lax.axis_index("c")
        s = lax.axis_index("s")
        w = s * NC + c
        base = w * nb_ent
        pltpu.sync_copy(idx_hbm.at[pl.ds(base, nb_ent)], ent_v)
        lanes = lax.iota(jnp.int32, L)

        def body(ch, _):
            ids = ent_v[pl.ds(ch * E, E)]
            ha = (
                pltpu.async_copy(conn_hbm.at[ids], conn_v, sem_a),
                pltpu.async_copy(deg_hbm.at[ids], deg_v, sem_a),
                pltpu.async_copy(emb_hbm.at[ids], self_v, sem_a),
            )
            for h in ha:
                h.wait()
            hn = []
            for kk in range(k):
                col = plsc.load_gather(conn_v, [lanes, jnp.full((L,), kk, jnp.int32)])
                hn.append(pltpu.async_copy(emb_hbm.at[col], nb_v.at[kk], sem_n))
            for h in hn:
                h.wait()
            for e in range(E):
                accs = tuple(self_v[e, pl.ds(L * j, L)] for j in range(nd))

                def kbody(t, accs, _e=e):
                    a = list(accs)
                    for u in range(UNROLL):
                        kk = t * UNROLL + u
                        for j in range(nd):
                            a[j] = a[j] + nb_v[kk, _e, pl.ds(L * j, L)]
                    return tuple(a)

                accs = lax.fori_loop(0, k // UNROLL, kbody, accs)
                rec = 1.0 / deg_v[e, pl.ds(0, L)].astype(jnp.float32)
                row = ch * E + e
                for j in range(nd):
                    out_all[row, pl.ds(L * j, L)] = accs[j] * rec
            return ()

        lax.fori_loop(0, nch, body, ())
        pltpu.sync_copy(out_all, out_hbm.at[pl.ds(base, nb_ent)])

    return onehop


def kernel(idx, connections, e1_degrees, embedding_weight):
    b, f, two = idx.shape
    num_ent, d = embedding_weight.shape
    k = connections.shape[1]
    ntot = b * f * two
    idx_flat = idx.reshape(ntot).astype(jnp.int32)
    deg2 = jnp.broadcast_to(e1_degrees.reshape(num_ent, 1).astype(jnp.int32), (num_ent, L))
    kp = (k + L - 1) // L * L
    conn_p = jnp.pad(connections.astype(jnp.int32), ((0, 0), (0, kp - k)))
    fn = _build(num_ent, d, k, ntot)
    out = fn(idx_flat, conn_p, deg2, embedding_weight)
    return out.reshape(b, f, two, d)


# speedup vs baseline: 17.2437x; 1.1932x over previous
import functools

import jax
import jax.numpy as jnp
from jax import lax
from jax.experimental import pallas as pl
from jax.experimental.pallas import tpu as pltpu
from jax.experimental.pallas import tpu_sc as plsc

NC = 2
NS = 16
NW = NC * NS
L = 16
E = 16
UNROLL = 5


@functools.lru_cache(maxsize=None)
def _build(num_ent, d, k, ntot):
    assert d % L == 0 and ntot % (NW * E) == 0 and k % UNROLL == 0
    kp = (k + L - 1) // L * L
    nd = d // L
    nb_ent = ntot // NW
    nch = nb_ent // E
    nq = nch // 2
    assert nch % 2 == 0 and nch >= 4

    mesh = plsc.VectorSubcoreMesh(core_axis_name="c", subcore_axis_name="s")

    @functools.partial(
        pl.kernel,
        out_type=jax.ShapeDtypeStruct((ntot, d), jnp.float32),
        mesh=mesh,
        compiler_params=pltpu.CompilerParams(
            use_tc_tiling_on_sc=False, needs_layout_passes=False),
        scratch_types=[
            pltpu.VMEM((nb_ent,), jnp.int32),
            pltpu.VMEM((2, E, kp), jnp.int32),
            pltpu.VMEM((2, E, L), jnp.int32),
            pltpu.VMEM((2, E, d), jnp.float32),
            pltpu.VMEM((2, k, E, d), jnp.float32),
            pltpu.VMEM((2, E, d), jnp.float32),
            pltpu.SemaphoreType.DMA,
            pltpu.SemaphoreType.DMA,
            pltpu.SemaphoreType.DMA,
            pltpu.SemaphoreType.DMA,
            pltpu.SemaphoreType.DMA,
            pltpu.SemaphoreType.DMA,
            pltpu.SemaphoreType.DMA,
            pltpu.SemaphoreType.DMA,
        ],
    )
    def onehop(idx_hbm, conn_hbm, deg_hbm, emb_hbm, out_hbm,
               ent_v, conn_v, deg_v, self_v, nb_v, out_v,
               sem_c0, sem_c1, sem_sd0, sem_sd1, sem_n0, sem_n1,
               sem_o0, sem_o1):
        sem_c = (sem_c0, sem_c1)
        sem_sd = (sem_sd0, sem_sd1)
        sem_n = (sem_n0, sem_n1)
        sem_o = (sem_o0, sem_o1)
        c = lax.axis_index("c")
        s = lax.axis_index("s")
        w = s * NC + c
        base = w * nb_ent
        pltpu.sync_copy(idx_hbm.at[pl.ds(base, nb_ent)], ent_v)
        lanes = lax.iota(jnp.int32, L)
        zvec = jnp.zeros((L,), jnp.int32)

        def ids(ch):
            return ent_v[pl.ds(ch * E, E)]

        def fire_conn(b, ch):
            pltpu.async_copy(conn_hbm.at[ids(ch)], conn_v.at[b], sem_c[b])

        def wait_conn(b):
            pltpu.make_async_copy(conn_hbm.at[zvec], conn_v.at[b], sem_c[b]).wait()

        def fire_sd(b, ch):
            i = ids(ch)
            pltpu.async_copy(deg_hbm.at[i], deg_v.at[b], sem_sd[b])
            pltpu.async_copy(emb_hbm.at[i], self_v.at[b], sem_sd[b])

        def wait_sd(b):
            pltpu.make_async_copy(deg_hbm.at[zvec], deg_v.at[b], sem_sd[b]).wait()
            pltpu.make_async_copy(emb_hbm.at[zvec], self_v.at[b], sem_sd[b]).wait()

        def fire_nb(b):
            bfull = jnp.full((L,), b, jnp.int32)

            def fnb(kk, _):
                col = plsc.load_gather(
                    conn_v, [bfull, lanes, jnp.full((L,), kk, jnp.int32)])
                pltpu.async_copy(emb_hbm.at[col], nb_v.at[b, kk], sem_n[b])
                return ()

            lax.fori_loop(0, k, fnb, ())

        def wait_nb(b):
            def wnb(kk, _):
                pltpu.make_async_copy(emb_hbm.at[zvec], nb_v.at[b, kk], sem_n[b]).wait()
                return ()

            lax.fori_loop(0, k, wnb, ())

        def wait_out(b, ch):
            pltpu.make_async_copy(
                out_v.at[b], out_hbm.at[pl.ds(base + ch * E, E)], sem_o[b]).wait()

        def compute(b, ch):
            for e in range(E):
                accs = tuple(self_v[b, e, pl.ds(L * j, L)] for j in range(nd))

                def kbody(t, accs, _e=e):
                    a = list(accs)
                    for u in range(UNROLL):
                        kk = t * UNROLL + u
                        for j in range(nd):
                            a[j] = a[j] + nb_v[b, kk, _e, pl.ds(L * j, L)]
                    return tuple(a)

                accs = lax.fori_loop(0, k // UNROLL, kbody, accs)
                rec = 1.0 / deg_v[b, e, pl.ds(0, L)].astype(jnp.float32)
                for j in range(nd):
                    out_v[b, e, pl.ds(L * j, L)] = accs[j] * rec
            pltpu.async_copy(
                out_v.at[b], out_hbm.at[pl.ds(base + ch * E, E)], sem_o[b])

        def body(j, _):
            c0 = 2 * j
            c1 = c0 + 1
            g = j < nq - 1

            wait_conn(1)
            fire_nb(1)

            @pl.when(g)
            def _():
                fire_conn(0, c0 + 2)

            @pl.when(j > 0)
            def _():
                wait_out(0, c0 - 2)

            wait_sd(0)
            wait_nb(0)
            compute(0, c0)

            @pl.when(g)
            def _():
                fire_sd(0, c0 + 2)
                wait_conn(0)
                fire_nb(0)
                fire_conn(1, c0 + 3)

            @pl.when(j > 0)
            def _():
                wait_out(1, c0 - 1)

            wait_sd(1)
            wait_nb(1)
            compute(1, c1)

            @pl.when(g)
            def _():
                fire_sd(1, c0 + 3)

            return ()

        fire_conn(0, 0)
        fire_sd(0, 0)
        fire_conn(1, 1)
        fire_sd(1, 1)
        wait_conn(0)
        fire_nb(0)

        lax.fori_loop(0, nq, body, ())

        wait_out(0, nch - 2)
        wait_out(1, nch - 1)

    return onehop


def kernel(idx, connections, e1_degrees, embedding_weight):
    b, f, two = idx.shape
    num_ent, d = embedding_weight.shape
    k = connections.shape[1]
    ntot = b * f * two
    idx_flat = idx.reshape(ntot).astype(jnp.int32)
    deg2 = jnp.broadcast_to(e1_degrees.reshape(num_ent, 1).astype(jnp.int32), (num_ent, L))
    kp = (k + L - 1) // L * L
    conn_p = jnp.pad(connections.astype(jnp.int32), ((0, 0), (0, kp - k)))
    fn = _build(num_ent, d, k, ntot)
    out = fn(idx_flat, conn_p, deg2, embedding_weight)
    return out.reshape(b, f, two, d)


# speedup vs baseline: 19.8569x; 1.1515x over previous
import functools

import jax
import jax.numpy as jnp
from jax import lax
from jax.experimental import pallas as pl
from jax.experimental.pallas import tpu as pltpu
from jax.experimental.pallas import tpu_sc as plsc

NC = 2
NS = 16
NW = NC * NS
L = 16
E = 16
UNROLL = 5


@functools.lru_cache(maxsize=None)
def _build(num_ent, d, k, ntot):
    assert d % L == 0 and ntot % (NW * E) == 0 and k % UNROLL == 0
    assert num_ent % L == 0
    kp = (k + L - 1) // L * L
    nd = d // L
    nb_ent = ntot // NW
    nch = nb_ent // E
    nq = nch // 2
    assert nch % 2 == 0 and nch >= 4

    mesh = plsc.VectorSubcoreMesh(core_axis_name="c", subcore_axis_name="s")

    @functools.partial(
        pl.kernel,
        out_type=jax.ShapeDtypeStruct((ntot, d), jnp.float32),
        mesh=mesh,
        compiler_params=pltpu.CompilerParams(
            use_tc_tiling_on_sc=False, needs_layout_passes=False),
        scratch_types=[
            pltpu.VMEM((nb_ent,), jnp.int32),
            pltpu.VMEM((2, E, kp), jnp.int32),
            pltpu.VMEM((2, E, L), jnp.int32),
            pltpu.VMEM((2, L), jnp.float32),
            pltpu.VMEM((2, E, d), jnp.float32),
            pltpu.VMEM((2, k, E, d), jnp.float32),
            pltpu.VMEM((2, E, d), jnp.float32),
            pltpu.SemaphoreType.DMA,
            pltpu.SemaphoreType.DMA,
            pltpu.SemaphoreType.DMA,
            pltpu.SemaphoreType.DMA,
            pltpu.SemaphoreType.DMA,
            pltpu.SemaphoreType.DMA,
            pltpu.SemaphoreType.DMA,
            pltpu.SemaphoreType.DMA,
        ],
    )
    def onehop(idx_hbm, conn_hbm, deg_hbm, emb_hbm, out_hbm,
               ent_v, conn_v, deg_v, rec_v, self_v, nb_v, out_v,
               sem_c0, sem_c1, sem_sd0, sem_sd1, sem_n0, sem_n1,
               sem_o0, sem_o1):
        sem_c = (sem_c0, sem_c1)
        sem_sd = (sem_sd0, sem_sd1)
        sem_n = (sem_n0, sem_n1)
        sem_o = (sem_o0, sem_o1)
        c = lax.axis_index("c")
        s = lax.axis_index("s")
        w = s * NC + c
        base = w * nb_ent
        pltpu.sync_copy(idx_hbm.at[pl.ds(base, nb_ent)], ent_v)
        lanes = lax.iota(jnp.int32, L)
        zvec = jnp.zeros((L,), jnp.int32)

        def ids(ch):
            return ent_v[pl.ds(ch * E, E)]

        def fire_conn(b, ch):
            pltpu.async_copy(conn_hbm.at[ids(ch)], conn_v.at[b], sem_c[b])

        def wait_conn(b):
            pltpu.make_async_copy(conn_hbm.at[zvec], conn_v.at[b], sem_c[b]).wait()

        def fire_sd(b, ch):
            i = ids(ch)
            pltpu.async_copy(deg_hbm.at[i >> 4], deg_v.at[b], sem_sd[b])
            pltpu.async_copy(emb_hbm.at[i], self_v.at[b], sem_sd[b])

        def wait_sd(b):
            pltpu.make_async_copy(deg_hbm.at[zvec], deg_v.at[b], sem_sd[b]).wait()
            pltpu.make_async_copy(emb_hbm.at[zvec], self_v.at[b], sem_sd[b]).wait()

        def fire_nb(b):
            bfull = jnp.full((L,), b, jnp.int32)

            def fnb(kk, _):
                col = plsc.load_gather(
                    conn_v, [bfull, lanes, jnp.full((L,), kk, jnp.int32)])
                pltpu.async_copy(emb_hbm.at[col], nb_v.at[b, kk], sem_n[b])
                return ()

            lax.fori_loop(0, k, fnb, ())

        def wait_nb(b):
            def wnb(kk, _):
                pltpu.make_async_copy(emb_hbm.at[zvec], nb_v.at[b, kk], sem_n[b]).wait()
                return ()

            lax.fori_loop(0, k, wnb, ())

        def wait_out(b, ch):
            pltpu.make_async_copy(
                out_v.at[b], out_hbm.at[pl.ds(base + ch * E, E)], sem_o[b]).wait()

        def compute(b, ch):
            bfull = jnp.full((L,), b, jnp.int32)
            i16 = ids(ch)
            degs = plsc.load_gather(deg_v, [bfull, lanes, i16 & 15])
            rec16 = 1.0 / degs.astype(jnp.float32)
            for e in range(E):
                accs = tuple(self_v[b, e, pl.ds(L * j, L)] for j in range(nd))

                def kbody(t, accs, _e=e):
                    a = list(accs)
                    for u in range(UNROLL):
                        kk = t * UNROLL + u
                        for j in range(nd):
                            a[j] = a[j] + nb_v[b, kk, _e, pl.ds(L * j, L)]
                    return tuple(a)

                accs = lax.fori_loop(0, k // UNROLL, kbody, accs)
                rec = lax.gather(
                    rec16, jnp.full((L, 1), e, jnp.int32),
                    dimension_numbers=lax.GatherDimensionNumbers(
                        offset_dims=(), collapsed_slice_dims=(0,),
                        start_index_map=(0,)),
                    slice_sizes=(1,),
                    mode=lax.GatherScatterMode.PROMISE_IN_BOUNDS)
                for j in range(nd):
                    out_v[b, e, pl.ds(L * j, L)] = accs[j] * rec
            pltpu.async_copy(
                out_v.at[b], out_hbm.at[pl.ds(base + ch * E, E)], sem_o[b])

        def body(j, _):
            c0 = 2 * j
            c1 = c0 + 1
            g = j < nq - 1

            wait_conn(1)
            fire_nb(1)

            @pl.when(g)
            def _():
                fire_conn(0, c0 + 2)

            @pl.when(j > 0)
            def _():
                wait_out(0, c0 - 2)

            wait_sd(0)
            wait_nb(0)
            compute(0, c0)

            @pl.when(g)
            def _():
                fire_sd(0, c0 + 2)
                wait_conn(0)
                fire_nb(0)
                fire_conn(1, c0 + 3)

            @pl.when(j > 0)
            def _():
                wait_out(1, c0 - 1)

            wait_sd(1)
            wait_nb(1)
            compute(1, c1)

            @pl.when(g)
            def _():
                fire_sd(1, c0 + 3)

            return ()

        fire_conn(0, 0)
        fire_sd(0, 0)
        fire_conn(1, 1)
        fire_sd(1, 1)
        wait_conn(0)
        fire_nb(0)

        lax.fori_loop(0, nq, body, ())

        wait_out(0, nch - 2)
        wait_out(1, nch - 1)

    return onehop


def kernel(idx, connections, e1_degrees, embedding_weight):
    b, f, two = idx.shape
    num_ent, d = embedding_weight.shape
    k = connections.shape[1]
    ntot = b * f * two
    idx_flat = idx.reshape(ntot).astype(jnp.int32)
    kp = (k + L - 1) // L * L
    conn_p = jnp.pad(connections.astype(jnp.int32), ((0, 0), (0, kp - k)))
    deg2 = e1_degrees.reshape(num_ent // L, L).astype(jnp.int32)
    fn = _build(num_ent, d, k, ntot)
    out = fn(idx_flat, conn_p, deg2, embedding_weight)
    return out.reshape(b, f, two, d)
